# baseline (speedup 1.0000x reference)
import functools

import jax
import jax.numpy as jnp
from jax import lax
from jax.experimental import pallas as pl
from jax.experimental.pallas import tpu as pltpu
from jax.experimental.pallas import tpu_sc as plsc

S = 64.0
MARGIN = 0.4

NC = 2
NS = 16
NW = NC * NS
L = 16
NBUF = 6
R = 8


def _make_sc_kernel(B, V):
    nchunks = V // R
    tmax = pl.cdiv(nchunks, NW)
    ngroups = B // L
    mesh = plsc.VectorSubcoreMesh(core_axis_name="c", subcore_axis_name="s")

    @functools.partial(
        pl.kernel,
        mesh=mesh,
        out_type=jax.ShapeDtypeStruct((V, B), jnp.float32),
        scratch_types=[
            pltpu.VMEM((B,), jnp.int32),
            pltpu.VMEM((NBUF, R, B), jnp.float32),
            pltpu.VMEM((NBUF, R, B), jnp.float32),
            pltpu.SemaphoreType.DMA((NBUF,)),
            pltpu.SemaphoreType.DMA((NBUF,)),
        ],
    )
    def sc_fn(xt_hbm, labels_hbm, out_hbm, lab_v, ibuf, obuf, isem, osem):
        wid = lax.axis_index("s") * NC + lax.axis_index("c")
        pltpu.sync_copy(labels_hbm, lab_v)
        neg = jnp.full((L,), -MARGIN * S, jnp.float32)
        my_t = jnp.where(wid < nchunks - NW * (tmax - 1), tmax, tmax - 1)

        def start_in(b, t):
            off = pl.multiple_of((wid + t * NW) * R, R)
            pltpu.async_copy(
                xt_hbm.at[pl.ds(off, R), :], ibuf.at[b], isem.at[b]
            )

        def wait(sem_b, dst):
            pltpu.make_async_copy(
                xt_hbm.at[pl.ds(0, R), :], dst, sem_b
            ).wait()

        for b in range(NBUF):
            start_in(b, b)

        @pl.loop(0, pl.cdiv(tmax, NBUF) * NBUF, step=NBUF)
        def _ring(g):
          for b in range(NBUF):
            t = g + b

            @pl.when(t < my_t)
            def _step():
                voff = (wid + t * NW) * R
                wait(isem.at[b], ibuf.at[b])

                @pl.when(t >= NBUF)
                def _():
                    wait(osem.at[b], obuf.at[b])

                @plsc.parallel_loop(0, ngroups, unroll=8)
                def _v(i):
                    sl = pl.ds(i * L, L)
                    labv = lab_v[sl]
                    for lr in range(R):
                        x = ibuf[b, lr, sl] * S
                        obuf[b, lr, sl] = jnp.where(
                            labv == voff + lr, x + neg, x
                        )

                off = pl.multiple_of(voff, R)
                pltpu.async_copy(
                    obuf.at[b], out_hbm.at[pl.ds(off, R), :], osem.at[b]
                )

                @pl.when(t + NBUF < my_t)
                def _next():
                    start_in(b, t + NBUF)

        for b in range(NBUF):
            wait(osem.at[b], obuf.at[b])

    return sc_fn


def kernel(logits, labels, embeddings):
    B, V = logits.shape
    sc_fn = _make_sc_kernel(B, V)
    out_t = sc_fn(logits.T, labels.astype(jnp.int32))
    return out_t.T


# speedup vs baseline: 2.6842x; 2.6842x over previous
import functools

import jax
import jax.numpy as jnp
from jax import lax
from jax.experimental import pallas as pl
from jax.experimental.pallas import tpu as pltpu
from jax.experimental.pallas import tpu_sc as plsc

S = 64.0
MARGIN = 0.4

NC = 2
NS = 16
NW = NC * NS
L = 16
NBUF = 6
R = 8


def _make_sc_kernel(B, V):
    nchunks = V // R
    tmax = pl.cdiv(nchunks, NW)
    ngroups = B // L
    mesh = plsc.VectorSubcoreMesh(core_axis_name="c", subcore_axis_name="s")

    @functools.partial(
        pl.kernel,
        mesh=mesh,
        out_type=jax.ShapeDtypeStruct((V, B), jnp.float32),
        scratch_types=[
            pltpu.VMEM((B,), jnp.int32),
            pltpu.VMEM((NBUF, R, B), jnp.float32),
            pltpu.VMEM((NBUF, R, B), jnp.float32),
            pltpu.SemaphoreType.DMA((NBUF,)),
            pltpu.SemaphoreType.DMA((NBUF,)),
        ],
    )
    def sc_fn(xt_hbm, labels_hbm, out_hbm, lab_v, ibuf, obuf, isem, osem):
        wid = lax.axis_index("s") * NC + lax.axis_index("c")
        pltpu.sync_copy(labels_hbm, lab_v)
        neg = jnp.full((L,), -MARGIN * S, jnp.float32)
        my_t = jnp.where(wid < nchunks - NW * (tmax - 1), tmax, tmax - 1)

        def start_in(b, t):
            off = pl.multiple_of((wid + t * NW) * R, R)
            pltpu.async_copy(
                xt_hbm.at[pl.ds(off, R), :], ibuf.at[b], isem.at[b]
            )

        def wait(sem_b, dst):
            pltpu.make_async_copy(
                xt_hbm.at[pl.ds(0, R), :], dst, sem_b
            ).wait()

        for b in range(NBUF):
            start_in(b, b)

        @pl.loop(0, pl.cdiv(tmax, NBUF) * NBUF, step=NBUF)
        def _ring(g):
          for b in range(NBUF):
            t = g + b

            @pl.when(t < my_t)
            def _step():
                voff = (wid + t * NW) * R
                wait(isem.at[b], ibuf.at[b])

                @pl.when(t >= NBUF)
                def _():
                    wait(osem.at[b], obuf.at[b])

                @plsc.parallel_loop(0, ngroups, unroll=2)
                def _v(i):
                    sl = pl.ds(i * L, L)
                    labv = lab_v[sl]
                    for lr in range(R):
                        x = ibuf[b, lr, sl] * S
                        obuf[b, lr, sl] = jnp.where(
                            labv == voff + lr, x + neg, x
                        )

                off = pl.multiple_of(voff, R)
                pltpu.async_copy(
                    obuf.at[b], out_hbm.at[pl.ds(off, R), :], osem.at[b]
                )

                @pl.when(t + NBUF < my_t)
                def _next():
                    start_in(b, t + NBUF)

        for b in range(NBUF):
            wait(osem.at[b], obuf.at[b])

    return sc_fn


def kernel(logits, labels, embeddings):
    B, V = logits.shape
    sc_fn = _make_sc_kernel(B, V)
    out_t = sc_fn(logits.T, labels.astype(jnp.int32))
    return out_t.T


# speedup vs baseline: 2.6916x; 1.0028x over previous
import functools

import jax
import jax.numpy as jnp
from jax import lax
from jax.experimental import pallas as pl
from jax.experimental.pallas import tpu as pltpu
from jax.experimental.pallas import tpu_sc as plsc

S = 64.0
MARGIN = 0.4

NC = 2
NS = 16
NW = NC * NS
L = 16
NBUF = 6
R = 8


def _make_sc_kernel(B, V):
    nchunks = V // R
    tmax = pl.cdiv(nchunks, NW)
    ngroups = B // L
    mesh = plsc.VectorSubcoreMesh(core_axis_name="c", subcore_axis_name="s")

    @functools.partial(
        pl.kernel,
        mesh=mesh,
        out_type=jax.ShapeDtypeStruct((V, B), jnp.float32),
        scratch_types=[
            pltpu.VMEM((B,), jnp.int32),
            pltpu.VMEM((NBUF, R, B), jnp.float32),
            pltpu.VMEM((NBUF, R, B), jnp.float32),
            pltpu.SemaphoreType.DMA((NBUF,)),
            pltpu.SemaphoreType.DMA((NBUF,)),
        ],
    )
    def sc_fn(xt_hbm, labels_hbm, out_hbm, lab_v, ibuf, obuf, isem, osem):
        wid = lax.axis_index("s") * NC + lax.axis_index("c")
        pltpu.sync_copy(labels_hbm, lab_v)
        neg = jnp.full((L,), -MARGIN * S, jnp.float32)
        my_t = jnp.where(wid < nchunks - NW * (tmax - 1), tmax, tmax - 1)

        def start_in(b, t):
            off = pl.multiple_of((wid + t * NW) * R, R)
            pltpu.async_copy(
                xt_hbm.at[pl.ds(off, R), :], ibuf.at[b], isem.at[b]
            )

        def wait(sem_b, dst):
            pltpu.make_async_copy(
                xt_hbm.at[pl.ds(0, R), :], dst, sem_b
            ).wait()

        for b in range(NBUF):
            start_in(b, b)

        @pl.loop(0, pl.cdiv(tmax, NBUF) * NBUF, step=NBUF)
        def _ring(g):
          for b in range(NBUF):
            t = g + b

            @pl.when(t < my_t)
            def _step():
                voff = (wid + t * NW) * R
                wait(isem.at[b], ibuf.at[b])

                @pl.when(t >= NBUF)
                def _():
                    wait(osem.at[b], obuf.at[b])

                @plsc.parallel_loop(0, ngroups, unroll=1)
                def _v(i):
                    sl = pl.ds(i * L, L)
                    labv = lab_v[sl]
                    for lr in range(R):
                        x = ibuf[b, lr, sl] * S
                        obuf[b, lr, sl] = jnp.where(
                            labv == voff + lr, x + neg, x
                        )

                off = pl.multiple_of(voff, R)
                pltpu.async_copy(
                    obuf.at[b], out_hbm.at[pl.ds(off, R), :], osem.at[b]
                )

                @pl.when(t + NBUF < my_t)
                def _next():
                    start_in(b, t + NBUF)

        for b in range(NBUF):
            wait(osem.at[b], obuf.at[b])

    return sc_fn


def kernel(logits, labels, embeddings):
    B, V = logits.shape
    sc_fn = _make_sc_kernel(B, V)
    out_t = sc_fn(logits.T, labels.astype(jnp.int32))
    return out_t.T
